# Initial kernel scaffold; baseline (speedup 1.0000x reference)
import jax
import jax.numpy as jnp
from jax.experimental import pallas as pl


def kernel(x, params):
    raise NotImplementedError("write your pallas kernel here")


# speedup vs baseline: 3.9494x; 3.9494x over previous
import functools

import jax
import jax.numpy as jnp
from jax.experimental import pallas as pl

EPS = 1e-6
KGF = 20
_INF = float("inf")


def _select_topk_cols(dist, k):
    bm, n = dist.shape
    iota = jax.lax.broadcasted_iota(jnp.int32, (bm, n), 1)
    idxs = []
    vals = []
    for _ in range(k):
        mv = jnp.min(dist, axis=1, keepdims=True)
        aj = jnp.min(jnp.where(dist == mv, iota, n), axis=1)
        idxs.append(aj)
        vals.append(mv[:, 0])
        dist = jnp.where(iota == aj[:, None], _INF, dist)
    return idxs, vals


def _topk_mm_kernel(q_ref, r_ref, idx_ref, *, k):
    q = q_ref[...]
    r = r_ref[...]
    qr = jax.lax.dot_general(q, r, (((1,), (1,)), ((), ())),
                             preferred_element_type=jnp.float32)
    sqq = jnp.sum(q * q, axis=1, keepdims=True)
    sqr = jnp.sum(r * r, axis=1)[None, :]
    dist = sqq - 2.0 * qr
    dist = dist + sqr
    idxs, _ = _select_topk_cols(dist, k)
    idx_ref[...] = jnp.stack(idxs, axis=1)


def _topk_mm(f, k):
    n, d = f.shape
    bm = min(n, 256)
    return pl.pallas_call(
        functools.partial(_topk_mm_kernel, k=k),
        grid=(n // bm,),
        in_specs=[pl.BlockSpec((bm, d), lambda i: (i, 0)),
                  pl.BlockSpec((n, d), lambda i: (0, 0))],
        out_specs=pl.BlockSpec((bm, k), lambda i: (i, 0)),
        out_shape=jax.ShapeDtypeStruct((n, k), jnp.int32),
    )(f, f)


def _xyz_dist(q, rt):
    dx = q[:, 0:1] - rt[0:1, :]
    dy = q[:, 1:2] - rt[1:2, :]
    dz = q[:, 2:3] - rt[2:3, :]
    return (dx * dx + dy * dy) + dz * dz


def _topk_xyz_kernel(q_ref, rt_ref, idx_ref, *, k):
    dist = _xyz_dist(q_ref[...], rt_ref[...])
    idxs, _ = _select_topk_cols(dist, k)
    idx_ref[...] = jnp.stack(idxs, axis=1)


def _topk_xyz(q, r, k):
    m = q.shape[0]
    n = r.shape[0]
    bm = min(m, 512)
    return pl.pallas_call(
        functools.partial(_topk_xyz_kernel, k=k),
        grid=(m // bm,),
        in_specs=[pl.BlockSpec((bm, 3), lambda i: (i, 0)),
                  pl.BlockSpec((3, n), lambda i: (0, 0))],
        out_specs=pl.BlockSpec((bm, k), lambda i: (i, 0)),
        out_shape=jax.ShapeDtypeStruct((m, k), jnp.int32),
    )(q, r.T)


def _interp_kernel(q_ref, rt_ref, idx_ref, d_ref):
    dist = _xyz_dist(q_ref[...], rt_ref[...])
    idxs, vals = _select_topk_cols(dist, 3)
    idx_ref[...] = jnp.stack(idxs, axis=1)
    d_ref[...] = jnp.stack(vals, axis=1)


def _interp_nn3(p_dst, p_src):
    m = p_dst.shape[0]
    n = p_src.shape[0]
    bm = min(m, 512)
    return pl.pallas_call(
        _interp_kernel,
        grid=(m // bm,),
        in_specs=[pl.BlockSpec((bm, 3), lambda i: (i, 0)),
                  pl.BlockSpec((3, n), lambda i: (0, 0))],
        out_specs=[pl.BlockSpec((bm, 3), lambda i: (i, 0)),
                   pl.BlockSpec((bm, 3), lambda i: (i, 0))],
        out_shape=[jax.ShapeDtypeStruct((m, 3), jnp.int32),
                   jax.ShapeDtypeStruct((m, 3), jnp.float32)],
    )(p_dst, p_src.T)


def _fps_kernel(pt_ref, out_ref, *, m, n):
    ns = n // 128
    ms = m // 128
    px = pt_ref[0]
    py = pt_ref[1]
    pz = pt_ref[2]
    ii = (jax.lax.broadcasted_iota(jnp.int32, (ns, 128), 0) * 128
          + jax.lax.broadcasted_iota(jnp.int32, (ns, 128), 1))
    oi = (jax.lax.broadcasted_iota(jnp.int32, (ms, 128), 0) * 128
          + jax.lax.broadcasted_iota(jnp.int32, (ms, 128), 1))
    sel0 = ii == 0
    lx0 = jnp.sum(jnp.where(sel0, px, 0.0))
    ly0 = jnp.sum(jnp.where(sel0, py, 0.0))
    lz0 = jnp.sum(jnp.where(sel0, pz, 0.0))
    zo = jnp.zeros((ms, 128), jnp.float32)
    om0 = oi == 0
    ox0 = jnp.where(om0, lx0, zo)
    oy0 = jnp.where(om0, ly0, zo)
    oz0 = jnp.where(om0, lz0, zo)

    def body(i, st):
        dists, lx, ly, lz, ox, oy, oz = st
        dxx = px - lx
        dyy = py - ly
        dzz = pz - lz
        d = (dxx * dxx + dyy * dyy) + dzz * dzz
        dists = jnp.minimum(dists, d)
        mx = jnp.max(dists)
        nxt = jnp.min(jnp.where(dists == mx, ii, n))
        sel = ii == nxt
        nlx = jnp.sum(jnp.where(sel, px, 0.0))
        nly = jnp.sum(jnp.where(sel, py, 0.0))
        nlz = jnp.sum(jnp.where(sel, pz, 0.0))
        om = oi == i
        ox = jnp.where(om, nlx, ox)
        oy = jnp.where(om, nly, oy)
        oz = jnp.where(om, nlz, oz)
        return (dists, nlx, nly, nlz, ox, oy, oz)

    init = (jnp.full((ns, 128), _INF), lx0, ly0, lz0, ox0, oy0, oz0)
    dists, lx, ly, lz, ox, oy, oz = jax.lax.fori_loop(1, m, body, init)
    out_ref[0] = ox
    out_ref[1] = oy
    out_ref[2] = oz


def _fps_points(p, m):
    n = p.shape[0]
    pt = p.T.reshape(3, n // 128, 128)
    out = pl.pallas_call(
        functools.partial(_fps_kernel, m=m, n=n),
        out_shape=jax.ShapeDtypeStruct((3, m // 128, 128), jnp.float32),
    )(pt)
    return out.reshape(3, m).T


def _vn_batchnorm(x):
    norm = jnp.linalg.norm(x, axis=2) + EPS
    axes = (0,) + tuple(range(2, norm.ndim))
    mean = norm.mean(axis=axes, keepdims=True)
    var = norm.var(axis=axes, keepdims=True)
    norm_bn = (norm - mean) / jnp.sqrt(var + 1e-5)
    return x / jnp.expand_dims(norm, 2) * jnp.expand_dims(norm_bn, 2)


def _vnllr(x, Wf, Wd, negative_slope=0.2):
    p = jnp.einsum('oc,bc...->bo...', Wf, x)
    p = _vn_batchnorm(p)
    d = jnp.einsum('oc,bc...->bo...', Wd, x)
    dot = (p * d).sum(axis=2, keepdims=True)
    mask = (dot >= 0).astype(x.dtype)
    dns = (d * d).sum(axis=2, keepdims=True)
    return negative_slope * p + (1.0 - negative_slope) * (
        mask * p + (1.0 - mask) * (p - (dot / (dns + EPS)) * d))


def _graph_feature(x, k=KGF):
    b, c, _, n = x.shape
    f = x.reshape(c * 3, n).T
    idx = _topk_mm(f, k)
    feat = f[idx].reshape(n, k, c, 3)
    xc = f.reshape(n, 1, c, 3)
    out = jnp.concatenate([feat - xc, jnp.broadcast_to(xc, (n, k, c, 3))],
                          axis=2)
    return jnp.transpose(out, (2, 3, 0, 1))[None]


def _knn_group(nsample, p, n_p, x):
    idx = _topk_xyz(n_p, p, nsample)
    return x[:, :, idx][None]


def _interpolation(p_src, p_dst, feat):
    idx, dd = _interp_nn3(p_dst, p_src)
    recip = 1.0 / (jnp.maximum(dd, 0.0) + 1e-8)
    w = recip / recip.sum(-1, keepdims=True)
    f = feat[:, :, idx]
    return (f * w[None, None, :, :]).sum(-1)


def _transition_down(p, xf, Wf, Wd, stride, nsample):
    m = p.shape[0] // stride
    n_p = _fps_points(p, m)
    g = _knn_group(nsample, p, n_p, xf)
    h = _vnllr(g, Wf, Wd)
    return n_p, h.mean(-1)


def _transition_up(pA, xA, pB, xB, Wf1, Wd1, Wf2, Wd2):
    a = _vnllr(xA, Wf1, Wd1)
    b = _vnllr(xB, Wf2, Wd2)[0]
    return a + _interpolation(pB, pA, b)[None]


def _forward(x, params):
    P = params
    xt = jnp.transpose(x, (0, 2, 1))
    p1 = x[0]
    x1 = xt[:, None, :, :]
    x1 = _vnllr(_graph_feature(x1), P['conv1_Wf'], P['conv1_Wd']).mean(-1)
    p2, x2 = _transition_down(p1, x1[0], P['ds1_Wf'], P['ds1_Wd'], 2, 16)
    x2 = _vnllr(_graph_feature(x2), P['conv2_Wf'], P['conv2_Wd']).mean(-1)
    p3, x3 = _transition_down(p2, x2[0], P['ds2_Wf'], P['ds2_Wd'], 2, 16)
    x3 = _vnllr(_graph_feature(x3), P['conv3_Wf'], P['conv3_Wd']).mean(-1)
    p4, x4 = _transition_down(p3, x3[0], P['ds3_Wf'], P['ds3_Wd'], 2, 16)
    x4 = _vnllr(_graph_feature(x4), P['conv4_Wf'], P['conv4_Wd']).mean(-1)
    x4 = _vnllr(_graph_feature(x4), P['conv5_Wf'], P['conv5_Wd']).mean(-1)
    x5 = _transition_up(p3, x3, p4, x4, P['up1_m1_Wf'], P['up1_m1_Wd'],
                        P['up1_m2_Wf'], P['up1_m2_Wd'])
    x5 = _vnllr(_graph_feature(x5), P['conv6_Wf'], P['conv6_Wd']).mean(-1)
    x6 = _transition_up(p2, x2, p3, x5, P['up2_m1_Wf'], P['up2_m1_Wd'],
                        P['up2_m2_Wf'], P['up2_m2_Wd'])
    x6 = _vnllr(_graph_feature(x6), P['conv7_Wf'], P['conv7_Wd']).mean(-1)
    x7 = _transition_up(p1, x1, p2, x6, P['up3_m1_Wf'], P['up3_m1_Wd'],
                        P['up3_m2_Wf'], P['up3_m2_Wd'])
    x7 = _vnllr(_graph_feature(x7), P['conv8_Wf'], P['conv8_Wd']).mean(-1)
    return _vnllr(x7, P['conv9_Wf'], P['conv9_Wd'])


def kernel(x, params):
    return _forward(x, params)


# speedup vs baseline: 5.1190x; 1.2961x over previous
import functools

import jax
import jax.numpy as jnp
from jax.experimental import pallas as pl

EPS = 1e-6
KGF = 20
_INF = float("inf")


def _select_topk_cols(dist, k):
    bm, n = dist.shape
    iota = jax.lax.broadcasted_iota(jnp.int32, (bm, n), 1)
    idxs = []
    vals = []
    for _ in range(k):
        mv = jnp.min(dist, axis=1, keepdims=True)
        aj = jnp.min(jnp.where(dist == mv, iota, n), axis=1)
        idxs.append(aj)
        vals.append(mv[:, 0])
        dist = jnp.where(iota == aj[:, None], _INF, dist)
    return idxs, vals


def _topk_mm_kernel(q_ref, r_ref, idx_ref, *, k):
    q = q_ref[...]
    r = r_ref[...]
    qr = jax.lax.dot_general(q, r, (((1,), (1,)), ((), ())),
                             preferred_element_type=jnp.float32)
    sqq = jnp.sum(q * q, axis=1, keepdims=True)
    sqr = jnp.sum(r * r, axis=1)[None, :]
    dist = sqq - 2.0 * qr
    dist = dist + sqr
    idxs, _ = _select_topk_cols(dist, k)
    idx_ref[...] = jnp.stack(idxs, axis=1)


def _topk_mm(f, k):
    n, d = f.shape
    bm = min(n, 256)
    return pl.pallas_call(
        functools.partial(_topk_mm_kernel, k=k),
        grid=(n // bm,),
        in_specs=[pl.BlockSpec((bm, d), lambda i: (i, 0)),
                  pl.BlockSpec((n, d), lambda i: (0, 0))],
        out_specs=pl.BlockSpec((bm, k), lambda i: (i, 0)),
        out_shape=jax.ShapeDtypeStruct((n, k), jnp.int32),
    )(f, f)


def _xyz_dist(q, rt):
    dx = q[:, 0:1] - rt[0:1, :]
    dy = q[:, 1:2] - rt[1:2, :]
    dz = q[:, 2:3] - rt[2:3, :]
    return (dx * dx + dy * dy) + dz * dz


def _topk_xyz_kernel(q_ref, rt_ref, idx_ref, *, k):
    dist = _xyz_dist(q_ref[...], rt_ref[...])
    idxs, _ = _select_topk_cols(dist, k)
    idx_ref[...] = jnp.stack(idxs, axis=1)


def _topk_xyz(q, r, k):
    m = q.shape[0]
    n = r.shape[0]
    bm = min(m, 512)
    return pl.pallas_call(
        functools.partial(_topk_xyz_kernel, k=k),
        grid=(m // bm,),
        in_specs=[pl.BlockSpec((bm, 3), lambda i: (i, 0)),
                  pl.BlockSpec((3, n), lambda i: (0, 0))],
        out_specs=pl.BlockSpec((bm, k), lambda i: (i, 0)),
        out_shape=jax.ShapeDtypeStruct((m, k), jnp.int32),
    )(q, r.T)


def _interp_kernel(q_ref, rt_ref, idx_ref, d_ref):
    dist = _xyz_dist(q_ref[...], rt_ref[...])
    idxs, vals = _select_topk_cols(dist, 3)
    idx_ref[...] = jnp.stack(idxs, axis=1)
    d_ref[...] = jnp.stack(vals, axis=1)


def _interp_nn3(p_dst, p_src):
    m = p_dst.shape[0]
    n = p_src.shape[0]
    bm = min(m, 512)
    return pl.pallas_call(
        _interp_kernel,
        grid=(m // bm,),
        in_specs=[pl.BlockSpec((bm, 3), lambda i: (i, 0)),
                  pl.BlockSpec((3, n), lambda i: (0, 0))],
        out_specs=[pl.BlockSpec((bm, 3), lambda i: (i, 0)),
                   pl.BlockSpec((bm, 3), lambda i: (i, 0))],
        out_shape=[jax.ShapeDtypeStruct((m, 3), jnp.int32),
                   jax.ShapeDtypeStruct((m, 3), jnp.float32)],
    )(p_dst, p_src.T)


def _fps_kernel(pt_ref, out_ref, *, m, n):
    ns = n // 128
    ms = m // 128
    px = pt_ref[0]
    py = pt_ref[1]
    pz = pt_ref[2]
    ii = (jax.lax.broadcasted_iota(jnp.int32, (ns, 128), 0) * 128
          + jax.lax.broadcasted_iota(jnp.int32, (ns, 128), 1))
    oi = (jax.lax.broadcasted_iota(jnp.int32, (ms, 128), 0) * 128
          + jax.lax.broadcasted_iota(jnp.int32, (ms, 128), 1))
    sel0 = ii == 0
    lx0 = jnp.sum(jnp.where(sel0, px, 0.0))
    ly0 = jnp.sum(jnp.where(sel0, py, 0.0))
    lz0 = jnp.sum(jnp.where(sel0, pz, 0.0))
    zo = jnp.zeros((ms, 128), jnp.float32)
    om0 = oi == 0
    ox0 = jnp.where(om0, lx0, zo)
    oy0 = jnp.where(om0, ly0, zo)
    oz0 = jnp.where(om0, lz0, zo)

    def body(i, st):
        dists, lx, ly, lz, ox, oy, oz = st
        dxx = px - lx
        dyy = py - ly
        dzz = pz - lz
        d = (dxx * dxx + dyy * dyy) + dzz * dzz
        dists = jnp.minimum(dists, d)
        mx = jnp.max(dists)
        nxt = jnp.min(jnp.where(dists == mx, ii, n))
        sel = ii == nxt
        nlx = jnp.sum(jnp.where(sel, px, 0.0))
        nly = jnp.sum(jnp.where(sel, py, 0.0))
        nlz = jnp.sum(jnp.where(sel, pz, 0.0))
        om = oi == i
        ox = jnp.where(om, nlx, ox)
        oy = jnp.where(om, nly, oy)
        oz = jnp.where(om, nlz, oz)
        return (dists, nlx, nly, nlz, ox, oy, oz)

    init = (jnp.full((ns, 128), _INF), lx0, ly0, lz0, ox0, oy0, oz0)
    dists, lx, ly, lz, ox, oy, oz = jax.lax.fori_loop(1, m, body, init)
    out_ref[0] = ox
    out_ref[1] = oy
    out_ref[2] = oz


def _fps_points(p, m):
    n = p.shape[0]
    pt = p.T.reshape(3, n // 128, 128)
    out = pl.pallas_call(
        functools.partial(_fps_kernel, m=m, n=n),
        out_shape=jax.ShapeDtypeStruct((3, m // 128, 128), jnp.float32),
    )(pt)
    return out.reshape(3, m).T


def _vn_batchnorm(x):
    norm = jnp.linalg.norm(x, axis=2) + EPS
    axes = (0,) + tuple(range(2, norm.ndim))
    mean = norm.mean(axis=axes, keepdims=True)
    var = norm.var(axis=axes, keepdims=True)
    norm_bn = (norm - mean) / jnp.sqrt(var + 1e-5)
    return x / jnp.expand_dims(norm, 2) * jnp.expand_dims(norm_bn, 2)


def _vnllr(x, Wf, Wd, negative_slope=0.2):
    p = jnp.einsum('oc,bc...->bo...', Wf, x)
    p = _vn_batchnorm(p)
    d = jnp.einsum('oc,bc...->bo...', Wd, x)
    dot = (p * d).sum(axis=2, keepdims=True)
    mask = (dot >= 0).astype(x.dtype)
    dns = (d * d).sum(axis=2, keepdims=True)
    return negative_slope * p + (1.0 - negative_slope) * (
        mask * p + (1.0 - mask) * (p - (dot / (dns + EPS)) * d))


def _graph_feature(x, k=KGF):
    b, c, _, n = x.shape
    ft = x.reshape(c * 3, n)
    f = ft.T
    idx = _topk_mm(f, k)
    feat_t = jnp.transpose(f[idx], (2, 0, 1))
    xc_t = ft[:, :, None]
    diff = (feat_t - xc_t).reshape(c, 3, n, k)
    xcb = jnp.broadcast_to(xc_t, (c * 3, n, k)).reshape(c, 3, n, k)
    return jnp.concatenate([diff, xcb], axis=0)[None]


def _knn_group(nsample, p, n_p, x):
    idx = _topk_xyz(n_p, p, nsample)
    c, _, n = x.shape
    m = idx.shape[0]
    fr = x.reshape(c * 3, n).T
    g = fr[idx.reshape(-1)]
    return jnp.transpose(g.reshape(m, nsample, c * 3),
                         (2, 0, 1)).reshape(c, 3, m, nsample)[None]


def _interpolation(p_src, p_dst, feat):
    idx, dd = _interp_nn3(p_dst, p_src)
    recip = 1.0 / (jnp.maximum(dd, 0.0) + 1e-8)
    w = recip / recip.sum(-1, keepdims=True)
    c, _, n_src = feat.shape
    m = idx.shape[0]
    fr = feat.reshape(c * 3, n_src).T
    g = fr[idx.reshape(-1)]
    f = jnp.transpose(g.reshape(m, 3, c * 3), (2, 0, 1)).reshape(c, 3, m, 3)
    return (f * w[None, None, :, :]).sum(-1)


def _transition_down(p, xf, Wf, Wd, stride, nsample):
    m = p.shape[0] // stride
    n_p = _fps_points(p, m)
    g = _knn_group(nsample, p, n_p, xf)
    h = _vnllr(g, Wf, Wd)
    return n_p, h.mean(-1)


def _transition_up(pA, xA, pB, xB, Wf1, Wd1, Wf2, Wd2):
    a = _vnllr(xA, Wf1, Wd1)
    b = _vnllr(xB, Wf2, Wd2)[0]
    return a + _interpolation(pB, pA, b)[None]


def _forward(x, params):
    P = params
    xt = jnp.transpose(x, (0, 2, 1))
    p1 = x[0]
    x1 = xt[:, None, :, :]
    x1 = _vnllr(_graph_feature(x1), P['conv1_Wf'], P['conv1_Wd']).mean(-1)
    p2, x2 = _transition_down(p1, x1[0], P['ds1_Wf'], P['ds1_Wd'], 2, 16)
    x2 = _vnllr(_graph_feature(x2), P['conv2_Wf'], P['conv2_Wd']).mean(-1)
    p3, x3 = _transition_down(p2, x2[0], P['ds2_Wf'], P['ds2_Wd'], 2, 16)
    x3 = _vnllr(_graph_feature(x3), P['conv3_Wf'], P['conv3_Wd']).mean(-1)
    p4, x4 = _transition_down(p3, x3[0], P['ds3_Wf'], P['ds3_Wd'], 2, 16)
    x4 = _vnllr(_graph_feature(x4), P['conv4_Wf'], P['conv4_Wd']).mean(-1)
    x4 = _vnllr(_graph_feature(x4), P['conv5_Wf'], P['conv5_Wd']).mean(-1)
    x5 = _transition_up(p3, x3, p4, x4, P['up1_m1_Wf'], P['up1_m1_Wd'],
                        P['up1_m2_Wf'], P['up1_m2_Wd'])
    x5 = _vnllr(_graph_feature(x5), P['conv6_Wf'], P['conv6_Wd']).mean(-1)
    x6 = _transition_up(p2, x2, p3, x5, P['up2_m1_Wf'], P['up2_m1_Wd'],
                        P['up2_m2_Wf'], P['up2_m2_Wd'])
    x6 = _vnllr(_graph_feature(x6), P['conv7_Wf'], P['conv7_Wd']).mean(-1)
    x7 = _transition_up(p1, x1, p2, x6, P['up3_m1_Wf'], P['up3_m1_Wd'],
                        P['up3_m2_Wf'], P['up3_m2_Wd'])
    x7 = _vnllr(_graph_feature(x7), P['conv8_Wf'], P['conv8_Wd']).mean(-1)
    return _vnllr(x7, P['conv9_Wf'], P['conv9_Wd'])


def kernel(x, params):
    return _forward(x, params)


# speedup vs baseline: 5.2076x; 1.0173x over previous
import functools

import jax
import jax.numpy as jnp
from jax.experimental import pallas as pl
from jax.experimental.pallas import tpu as pltpu

EPS = 1e-6
KGF = 20
_INF = float("inf")


def _select_topk_cols(dist, k):
    bm, n = dist.shape
    iota = jax.lax.broadcasted_iota(jnp.int32, (bm, n), 1)
    idxs = []
    vals = []
    for _ in range(k):
        mv = jnp.min(dist, axis=1, keepdims=True)
        aj = jnp.min(jnp.where(dist == mv, iota, n), axis=1)
        idxs.append(aj)
        vals.append(mv[:, 0])
        dist = jnp.where(iota == aj[:, None], _INF, dist)
    return idxs, vals


def _sel_topk_kernel(dist_ref, idx_ref, *, k):
    idxs, _ = _select_topk_cols(dist_ref[...], k)
    idx_ref[...] = jnp.stack(idxs, axis=1)


def _sel_topk(dist, k):
    m, n = dist.shape
    bm = min(m, 256)
    return pl.pallas_call(
        functools.partial(_sel_topk_kernel, k=k),
        grid=(m // bm,),
        in_specs=[pl.BlockSpec((bm, n), lambda i: (i, 0))],
        out_specs=pl.BlockSpec((bm, k), lambda i: (i, 0)),
        out_shape=jax.ShapeDtypeStruct((m, k), jnp.int32),
    )(dist)


def _topk_mm(f, k):
    inner = f @ f.T
    sq = (f * f).sum(-1)
    dist = sq[:, None] - 2.0 * inner + sq[None, :]
    return _sel_topk(dist, k)


def _topk_xyz(q, r, k):
    dist = ((q[:, None, :] - r[None, :, :]) ** 2).sum(-1)
    return _sel_topk(dist, k)


def _sel_topk3v_kernel(dist_ref, idx_ref, d_ref):
    idxs, vals = _select_topk_cols(dist_ref[...], 3)
    idx_ref[...] = jnp.stack(idxs, axis=1)
    d_ref[...] = jnp.stack(vals, axis=1)


def _interp_nn3(p_dst, p_src):
    dist = ((p_dst[:, None, :] - p_src[None, :, :]) ** 2).sum(-1)
    m, n = dist.shape
    bm = min(m, 256)
    return pl.pallas_call(
        _sel_topk3v_kernel,
        grid=(m // bm,),
        in_specs=[pl.BlockSpec((bm, n), lambda i: (i, 0))],
        out_specs=[pl.BlockSpec((bm, 3), lambda i: (i, 0)),
                   pl.BlockSpec((bm, 3), lambda i: (i, 0))],
        out_shape=[jax.ShapeDtypeStruct((m, 3), jnp.int32),
                   jax.ShapeDtypeStruct((m, 3), jnp.float32)],
    )(dist)


def _fps_kernel(pt_ref, out_ref, *, m, n):
    ns = n // 128
    ms = m // 128
    px = pt_ref[0]
    py = pt_ref[1]
    pz = pt_ref[2]
    ii = (jax.lax.broadcasted_iota(jnp.int32, (ns, 128), 0) * 128
          + jax.lax.broadcasted_iota(jnp.int32, (ns, 128), 1))
    oi = (jax.lax.broadcasted_iota(jnp.int32, (ms, 128), 0) * 128
          + jax.lax.broadcasted_iota(jnp.int32, (ms, 128), 1))
    sel0 = ii == 0
    lx0 = jnp.sum(jnp.where(sel0, px, 0.0))
    ly0 = jnp.sum(jnp.where(sel0, py, 0.0))
    lz0 = jnp.sum(jnp.where(sel0, pz, 0.0))
    zo = jnp.zeros((ms, 128), jnp.float32)
    om0 = oi == 0
    ox0 = jnp.where(om0, lx0, zo)
    oy0 = jnp.where(om0, ly0, zo)
    oz0 = jnp.where(om0, lz0, zo)

    def body(i, st):
        dists, lx, ly, lz, ox, oy, oz = st
        dxx = px - lx
        dyy = py - ly
        dzz = pz - lz
        d = (dxx * dxx + dyy * dyy) + dzz * dzz
        dists = jnp.minimum(dists, d)
        mx = jnp.max(dists)
        nxt = jnp.min(jnp.where(dists == mx, ii, n))
        sel = ii == nxt
        nlx = jnp.sum(jnp.where(sel, px, 0.0))
        nly = jnp.sum(jnp.where(sel, py, 0.0))
        nlz = jnp.sum(jnp.where(sel, pz, 0.0))
        om = oi == i
        ox = jnp.where(om, nlx, ox)
        oy = jnp.where(om, nly, oy)
        oz = jnp.where(om, nlz, oz)
        return (dists, nlx, nly, nlz, ox, oy, oz)

    init = (jnp.full((ns, 128), _INF), lx0, ly0, lz0, ox0, oy0, oz0)
    dists, lx, ly, lz, ox, oy, oz = jax.lax.fori_loop(1, m, body, init)
    out_ref[0] = ox
    out_ref[1] = oy
    out_ref[2] = oz


def _fps_dmat_kernel(p_ref, pt_ref, sel_ref, d_ref, *, m, n):
    pv = p_ref[...]
    pt = pt_ref[...]
    dx = pv[:, 0:1] - pt[0:1, :]
    dy = pv[:, 1:2] - pt[1:2, :]
    dz = pv[:, 2:3] - pt[2:3, :]
    d_ref[...] = (dx * dx + dy * dy) + dz * dz
    ii = jax.lax.broadcasted_iota(jnp.int32, (1, n), 1)
    oi = jax.lax.broadcasted_iota(jnp.int32, (1, m), 1)

    def body(i, st):
        dists, last, sel = st
        dists = jnp.minimum(dists, d_ref[pl.ds(last, 1), :])
        mx = jnp.max(dists)
        nxt = jnp.min(jnp.where(dists == mx, ii, n))
        sel = jnp.where(oi == i, nxt, sel)
        return (dists, nxt, sel)

    init = (jnp.full((1, n), _INF), jnp.int32(0), jnp.zeros((1, m), jnp.int32))
    _, _, sel = jax.lax.fori_loop(1, m, body, init)
    sel_ref[...] = sel


def _fps_points(p, m):
    n = p.shape[0]
    pt = p.T.reshape(3, n // 128, 128)
    if n <= 2048:
        sel = pl.pallas_call(
            functools.partial(_fps_dmat_kernel, m=m, n=n),
            out_shape=jax.ShapeDtypeStruct((1, m), jnp.int32),
            scratch_shapes=[pltpu.VMEM((n, n), jnp.float32)],
        )(p, p.T)
        return p[sel.reshape(m)]
    out = pl.pallas_call(
        functools.partial(_fps_kernel, m=m, n=n),
        out_shape=jax.ShapeDtypeStruct((3, m // 128, 128), jnp.float32),
    )(pt)
    return out.reshape(3, m).T


def _vn_batchnorm(x):
    norm = jnp.linalg.norm(x, axis=2) + EPS
    axes = (0,) + tuple(range(2, norm.ndim))
    mean = norm.mean(axis=axes, keepdims=True)
    var = norm.var(axis=axes, keepdims=True)
    norm_bn = (norm - mean) / jnp.sqrt(var + 1e-5)
    return x / jnp.expand_dims(norm, 2) * jnp.expand_dims(norm_bn, 2)


def _vnllr(x, Wf, Wd, negative_slope=0.2):
    p = jnp.einsum('oc,bc...->bo...', Wf, x)
    p = _vn_batchnorm(p)
    d = jnp.einsum('oc,bc...->bo...', Wd, x)
    dot = (p * d).sum(axis=2, keepdims=True)
    mask = (dot >= 0).astype(x.dtype)
    dns = (d * d).sum(axis=2, keepdims=True)
    return negative_slope * p + (1.0 - negative_slope) * (
        mask * p + (1.0 - mask) * (p - (dot / (dns + EPS)) * d))


def _graph_feature(x, k=KGF):
    b, c, _, n = x.shape
    ft = x.reshape(c * 3, n)
    f = ft.T
    idx = _topk_mm(f, k)
    feat_t = jnp.transpose(f[idx], (2, 0, 1))
    xc_t = ft[:, :, None]
    diff = (feat_t - xc_t).reshape(c, 3, n, k)
    xcb = jnp.broadcast_to(xc_t, (c * 3, n, k)).reshape(c, 3, n, k)
    return jnp.concatenate([diff, xcb], axis=0)[None]


def _knn_group(nsample, p, n_p, x):
    idx = _topk_xyz(n_p, p, nsample)
    c, _, n = x.shape
    m = idx.shape[0]
    fr = x.reshape(c * 3, n).T
    g = fr[idx.reshape(-1)]
    return jnp.transpose(g.reshape(m, nsample, c * 3),
                         (2, 0, 1)).reshape(c, 3, m, nsample)[None]


def _interpolation(p_src, p_dst, feat):
    idx, dd = _interp_nn3(p_dst, p_src)
    recip = 1.0 / (jnp.maximum(dd, 0.0) + 1e-8)
    w = recip / recip.sum(-1, keepdims=True)
    c, _, n_src = feat.shape
    m = idx.shape[0]
    fr = feat.reshape(c * 3, n_src).T
    g = fr[idx.reshape(-1)]
    f = jnp.transpose(g.reshape(m, 3, c * 3), (2, 0, 1)).reshape(c, 3, m, 3)
    return (f * w[None, None, :, :]).sum(-1)


def _transition_down(p, xf, Wf, Wd, stride, nsample):
    m = p.shape[0] // stride
    n_p = _fps_points(p, m)
    g = _knn_group(nsample, p, n_p, xf)
    h = _vnllr(g, Wf, Wd)
    return n_p, h.mean(-1)


def _transition_up(pA, xA, pB, xB, Wf1, Wd1, Wf2, Wd2):
    a = _vnllr(xA, Wf1, Wd1)
    b = _vnllr(xB, Wf2, Wd2)[0]
    return a + _interpolation(pB, pA, b)[None]


def _forward(x, params):
    P = params
    xt = jnp.transpose(x, (0, 2, 1))
    p1 = x[0]
    x1 = xt[:, None, :, :]
    x1 = _vnllr(_graph_feature(x1), P['conv1_Wf'], P['conv1_Wd']).mean(-1)
    p2, x2 = _transition_down(p1, x1[0], P['ds1_Wf'], P['ds1_Wd'], 2, 16)
    x2 = _vnllr(_graph_feature(x2), P['conv2_Wf'], P['conv2_Wd']).mean(-1)
    p3, x3 = _transition_down(p2, x2[0], P['ds2_Wf'], P['ds2_Wd'], 2, 16)
    x3 = _vnllr(_graph_feature(x3), P['conv3_Wf'], P['conv3_Wd']).mean(-1)
    p4, x4 = _transition_down(p3, x3[0], P['ds3_Wf'], P['ds3_Wd'], 2, 16)
    x4 = _vnllr(_graph_feature(x4), P['conv4_Wf'], P['conv4_Wd']).mean(-1)
    x4 = _vnllr(_graph_feature(x4), P['conv5_Wf'], P['conv5_Wd']).mean(-1)
    x5 = _transition_up(p3, x3, p4, x4, P['up1_m1_Wf'], P['up1_m1_Wd'],
                        P['up1_m2_Wf'], P['up1_m2_Wd'])
    x5 = _vnllr(_graph_feature(x5), P['conv6_Wf'], P['conv6_Wd']).mean(-1)
    x6 = _transition_up(p2, x2, p3, x5, P['up2_m1_Wf'], P['up2_m1_Wd'],
                        P['up2_m2_Wf'], P['up2_m2_Wd'])
    x6 = _vnllr(_graph_feature(x6), P['conv7_Wf'], P['conv7_Wd']).mean(-1)
    x7 = _transition_up(p1, x1, p2, x6, P['up3_m1_Wf'], P['up3_m1_Wd'],
                        P['up3_m2_Wf'], P['up3_m2_Wd'])
    x7 = _vnllr(_graph_feature(x7), P['conv8_Wf'], P['conv8_Wd']).mean(-1)
    return _vnllr(x7, P['conv9_Wf'], P['conv9_Wd'])


def kernel(x, params):
    return _forward(x, params)


# speedup vs baseline: 5.4500x; 1.0465x over previous
import functools

import jax
import jax.numpy as jnp
from jax.experimental import pallas as pl
from jax.experimental.pallas import tpu as pltpu

EPS = 1e-6
KGF = 20
_INF = float("inf")


def _select_topk_cols(dist, k):
    bm, n = dist.shape
    iota = jax.lax.broadcasted_iota(jnp.int32, (bm, n), 1)
    idxs = []
    vals = []
    for _ in range(k):
        mv = jnp.min(dist, axis=1, keepdims=True)
        aj = jnp.min(jnp.where(dist == mv, iota, n), axis=1)
        idxs.append(aj)
        vals.append(mv[:, 0])
        dist = jnp.where(iota == aj[:, None], _INF, dist)
    return idxs, vals


def _sel_topk_kernel(dist_ref, idx_ref, *, k):
    idxs, _ = _select_topk_cols(dist_ref[...], k)
    idx_ref[...] = jnp.stack(idxs, axis=1)


def _sel_topk(dist, k):
    m, n = dist.shape
    bm = min(m, 256)
    return pl.pallas_call(
        functools.partial(_sel_topk_kernel, k=k),
        grid=(m // bm,),
        in_specs=[pl.BlockSpec((bm, n), lambda i: (i, 0))],
        out_specs=pl.BlockSpec((bm, k), lambda i: (i, 0)),
        out_shape=jax.ShapeDtypeStruct((m, k), jnp.int32),
    )(dist)


def _topk_mm(f, k):
    inner = f @ f.T
    sq = (f * f).sum(-1)
    dist = sq[:, None] - 2.0 * inner + sq[None, :]
    return _sel_topk(dist, k)


def _topk_xyz(q, r, k):
    dist = ((q[:, None, :] - r[None, :, :]) ** 2).sum(-1)
    return _sel_topk(dist, k)


def _sel_topk3v_kernel(dist_ref, idx_ref, d_ref):
    idxs, vals = _select_topk_cols(dist_ref[...], 3)
    idx_ref[...] = jnp.stack(idxs, axis=1)
    d_ref[...] = jnp.stack(vals, axis=1)


def _interp_nn3(p_dst, p_src):
    dist = ((p_dst[:, None, :] - p_src[None, :, :]) ** 2).sum(-1)
    m, n = dist.shape
    bm = min(m, 256)
    return pl.pallas_call(
        _sel_topk3v_kernel,
        grid=(m // bm,),
        in_specs=[pl.BlockSpec((bm, n), lambda i: (i, 0))],
        out_specs=[pl.BlockSpec((bm, 3), lambda i: (i, 0)),
                   pl.BlockSpec((bm, 3), lambda i: (i, 0))],
        out_shape=[jax.ShapeDtypeStruct((m, 3), jnp.int32),
                   jax.ShapeDtypeStruct((m, 3), jnp.float32)],
    )(dist)


def _fps_kernel(pt_ref, ps_ref, sel_ref, *, m, n):
    ns = n // 128
    ms = m // 128
    px = pt_ref[0]
    py = pt_ref[1]
    pz = pt_ref[2]
    ii = (jax.lax.broadcasted_iota(jnp.int32, (ns, 128), 0) * 128
          + jax.lax.broadcasted_iota(jnp.int32, (ns, 128), 1))
    oi = (jax.lax.broadcasted_iota(jnp.int32, (ms, 128), 0) * 128
          + jax.lax.broadcasted_iota(jnp.int32, (ms, 128), 1))

    def body(i, st):
        dists, last, sel = st
        lx = ps_ref[0, last]
        ly = ps_ref[1, last]
        lz = ps_ref[2, last]
        dxx = px - lx
        dyy = py - ly
        dzz = pz - lz
        d = (dxx * dxx + dyy * dyy) + dzz * dzz
        dists = jnp.minimum(dists, d)
        mx = jnp.max(dists)
        nxt = jnp.min(jnp.where(dists == mx, ii, n))
        sel = jnp.where(oi == i, nxt, sel)
        return (dists, nxt, sel)

    init = (jnp.full((ns, 128), _INF), jnp.int32(0),
            jnp.zeros((ms, 128), jnp.int32))
    _, _, sel = jax.lax.fori_loop(1, m, body, init)
    sel_ref[...] = sel


def _fps_dmat_kernel(p_ref, pt_ref, sel_ref, d_ref, *, m, n):
    pv = p_ref[...]
    pt = pt_ref[...]
    dx = pv[:, 0:1] - pt[0:1, :]
    dy = pv[:, 1:2] - pt[1:2, :]
    dz = pv[:, 2:3] - pt[2:3, :]
    d_ref[...] = (dx * dx + dy * dy) + dz * dz
    ii = jax.lax.broadcasted_iota(jnp.int32, (1, n), 1)
    oi = jax.lax.broadcasted_iota(jnp.int32, (1, m), 1)

    def body(i, st):
        dists, last, sel = st
        dists = jnp.minimum(dists, d_ref[pl.ds(last, 1), :])
        mx = jnp.max(dists)
        nxt = jnp.min(jnp.where(dists == mx, ii, n))
        sel = jnp.where(oi == i, nxt, sel)
        return (dists, nxt, sel)

    init = (jnp.full((1, n), _INF), jnp.int32(0), jnp.zeros((1, m), jnp.int32))
    _, _, sel = jax.lax.fori_loop(1, m, body, init)
    sel_ref[...] = sel


def _fps_points(p, m):
    n = p.shape[0]
    if n <= 2048:
        sel = pl.pallas_call(
            functools.partial(_fps_dmat_kernel, m=m, n=n),
            out_shape=jax.ShapeDtypeStruct((1, m), jnp.int32),
            scratch_shapes=[pltpu.VMEM((n, n), jnp.float32)],
        )(p, p.T)
        return p[sel.reshape(m)]
    pt = p.T.reshape(3, n // 128, 128)
    sel = pl.pallas_call(
        functools.partial(_fps_kernel, m=m, n=n),
        in_specs=[pl.BlockSpec(memory_space=pltpu.MemorySpace.VMEM),
                  pl.BlockSpec(memory_space=pltpu.MemorySpace.SMEM)],
        out_shape=jax.ShapeDtypeStruct((m // 128, 128), jnp.int32),
    )(pt, p.T)
    return p[sel.reshape(m)]


def _vn_batchnorm(x):
    norm = jnp.linalg.norm(x, axis=2) + EPS
    axes = (0,) + tuple(range(2, norm.ndim))
    mean = norm.mean(axis=axes, keepdims=True)
    var = norm.var(axis=axes, keepdims=True)
    norm_bn = (norm - mean) / jnp.sqrt(var + 1e-5)
    return x / jnp.expand_dims(norm, 2) * jnp.expand_dims(norm_bn, 2)


def _vnllr(x, Wf, Wd, negative_slope=0.2):
    p = jnp.einsum('oc,bc...->bo...', Wf, x)
    p = _vn_batchnorm(p)
    d = jnp.einsum('oc,bc...->bo...', Wd, x)
    dot = (p * d).sum(axis=2, keepdims=True)
    mask = (dot >= 0).astype(x.dtype)
    dns = (d * d).sum(axis=2, keepdims=True)
    return negative_slope * p + (1.0 - negative_slope) * (
        mask * p + (1.0 - mask) * (p - (dot / (dns + EPS)) * d))


def _graph_feature(x, k=KGF):
    b, c, _, n = x.shape
    ft = x.reshape(c * 3, n)
    f = ft.T
    idx = _topk_mm(f, k)
    feat_t = jnp.transpose(f[idx], (2, 0, 1))
    xc_t = ft[:, :, None]
    diff = (feat_t - xc_t).reshape(c, 3, n, k)
    xcb = jnp.broadcast_to(xc_t, (c * 3, n, k)).reshape(c, 3, n, k)
    return jnp.concatenate([diff, xcb], axis=0)[None]


def _knn_group(nsample, p, n_p, x):
    idx = _topk_xyz(n_p, p, nsample)
    c, _, n = x.shape
    m = idx.shape[0]
    fr = x.reshape(c * 3, n).T
    g = fr[idx.reshape(-1)]
    return jnp.transpose(g.reshape(m, nsample, c * 3),
                         (2, 0, 1)).reshape(c, 3, m, nsample)[None]


def _interpolation(p_src, p_dst, feat):
    idx, dd = _interp_nn3(p_dst, p_src)
    recip = 1.0 / (jnp.maximum(dd, 0.0) + 1e-8)
    w = recip / recip.sum(-1, keepdims=True)
    c, _, n_src = feat.shape
    m = idx.shape[0]
    fr = feat.reshape(c * 3, n_src).T
    g = fr[idx.reshape(-1)]
    f = jnp.transpose(g.reshape(m, 3, c * 3), (2, 0, 1)).reshape(c, 3, m, 3)
    return (f * w[None, None, :, :]).sum(-1)


def _transition_down(p, xf, Wf, Wd, stride, nsample):
    m = p.shape[0] // stride
    n_p = _fps_points(p, m)
    g = _knn_group(nsample, p, n_p, xf)
    h = _vnllr(g, Wf, Wd)
    return n_p, h.mean(-1)


def _transition_up(pA, xA, pB, xB, Wf1, Wd1, Wf2, Wd2):
    a = _vnllr(xA, Wf1, Wd1)
    b = _vnllr(xB, Wf2, Wd2)[0]
    return a + _interpolation(pB, pA, b)[None]


def _forward(x, params):
    P = params
    xt = jnp.transpose(x, (0, 2, 1))
    p1 = x[0]
    x1 = xt[:, None, :, :]
    x1 = _vnllr(_graph_feature(x1), P['conv1_Wf'], P['conv1_Wd']).mean(-1)
    p2, x2 = _transition_down(p1, x1[0], P['ds1_Wf'], P['ds1_Wd'], 2, 16)
    x2 = _vnllr(_graph_feature(x2), P['conv2_Wf'], P['conv2_Wd']).mean(-1)
    p3, x3 = _transition_down(p2, x2[0], P['ds2_Wf'], P['ds2_Wd'], 2, 16)
    x3 = _vnllr(_graph_feature(x3), P['conv3_Wf'], P['conv3_Wd']).mean(-1)
    p4, x4 = _transition_down(p3, x3[0], P['ds3_Wf'], P['ds3_Wd'], 2, 16)
    x4 = _vnllr(_graph_feature(x4), P['conv4_Wf'], P['conv4_Wd']).mean(-1)
    x4 = _vnllr(_graph_feature(x4), P['conv5_Wf'], P['conv5_Wd']).mean(-1)
    x5 = _transition_up(p3, x3, p4, x4, P['up1_m1_Wf'], P['up1_m1_Wd'],
                        P['up1_m2_Wf'], P['up1_m2_Wd'])
    x5 = _vnllr(_graph_feature(x5), P['conv6_Wf'], P['conv6_Wd']).mean(-1)
    x6 = _transition_up(p2, x2, p3, x5, P['up2_m1_Wf'], P['up2_m1_Wd'],
                        P['up2_m2_Wf'], P['up2_m2_Wd'])
    x6 = _vnllr(_graph_feature(x6), P['conv7_Wf'], P['conv7_Wd']).mean(-1)
    x7 = _transition_up(p1, x1, p2, x6, P['up3_m1_Wf'], P['up3_m1_Wd'],
                        P['up3_m2_Wf'], P['up3_m2_Wd'])
    x7 = _vnllr(_graph_feature(x7), P['conv8_Wf'], P['conv8_Wd']).mean(-1)
    return _vnllr(x7, P['conv9_Wf'], P['conv9_Wd'])


def kernel(x, params):
    return _forward(x, params)
